# Initial kernel scaffold; baseline (speedup 1.0000x reference)
import jax
import jax.numpy as jnp
from jax.experimental import pallas as pl


def kernel(x_white, x_black, piece_count, emb, bias1, fc2_w, fc2_b, cp_w, cp_b, wdl_w, wdl_b):
    raise NotImplementedError("write your pallas kernel here")


# speedup vs baseline: 1.4497x; 1.4497x over previous
import functools

import jax
import jax.numpy as jnp
from jax import lax
from jax.experimental import pallas as pl
from jax.experimental.pallas import tpu as pltpu
from jax.experimental.pallas import tpu_sc as plsc

_B = 16384
_L = 32
_H = 1024
_NB = 8


def _make_sc_bag_sum(bags, L, H, interpret=False):
    nw = 32
    bpw = bags // nw
    ch = H // 16
    stage_rows = 8
    mesh = plsc.VectorSubcoreMesh(core_axis_name="c", subcore_axis_name="s",
                                  num_cores=2, num_subcores=16)

    @functools.partial(
        pl.kernel,
        out_type=jax.ShapeDtypeStruct((bags, H), jnp.float32),
        mesh=mesh,
        scratch_types=[
            pltpu.VMEM((bpw * L,), jnp.int32),
            pltpu.VMEM((L, H), jnp.float32),
            pltpu.VMEM((stage_rows, H), jnp.float32),
            pltpu.SemaphoreType.DMA,
        ],
        interpret=interpret,
    )
    def sc_bag_sum(x_hbm, emb_hbm, out_hbm, idx_v, gbuf, stage, sem):
        wid = lax.axis_index("s") * 2 + lax.axis_index("c")
        base = wid * bpw
        pltpu.sync_copy(
            x_hbm.at[pl.ds(pl.multiple_of(base * L, 8), bpw * L)], idx_v)

        def bag_body(i, carry):
            idx_row = idx_v.at[pl.ds(i * L, L)]
            pltpu.async_copy(emb_hbm.at[idx_row], gbuf, sem).wait()
            slot = lax.rem(i, stage_rows)

            def chunk_body(c, carry2):
                c16 = c * 16
                acc = gbuf[0, pl.ds(c16, 16)]
                for r in range(1, L):
                    acc = acc + gbuf[r, pl.ds(c16, 16)]
                stage[slot, pl.ds(c16, 16)] = acc
                return carry2

            lax.fori_loop(0, ch, chunk_body, 0)

            @pl.when(slot == stage_rows - 1)
            def _flush():
                off = pl.multiple_of(base + i - (stage_rows - 1), stage_rows)
                pltpu.sync_copy(stage, out_hbm.at[pl.ds(off, stage_rows)])

            return carry

        lax.fori_loop(0, bpw, bag_body, 0)

    return sc_bag_sum


def _screlu(x):
    return jnp.clip(x, 0.0, 1.0) ** 2


def _dense_body(hw_ref, hb_ref, pc_ref, b1_ref, w1t_ref, w2t_ref, b2_ref,
                cpwt_ref, cpb_ref, wdlwt_ref, wdlb_ref, cp_ref, wdl_ref):
    blk = hw_ref.shape[0]
    hw = _screlu(hw_ref[...] + b1_ref[...])
    hb = _screlu(hb_ref[...] + b1_ref[...])
    z = (jnp.dot(hw, w1t_ref[...], preferred_element_type=jnp.float32)
         + jnp.dot(hb, w2t_ref[...], preferred_element_type=jnp.float32)
         + b2_ref[...])
    h2 = _screlu(z)
    cp_all = jnp.dot(h2, cpwt_ref[...], preferred_element_type=jnp.float32) + cpb_ref[...]
    wdl_all = jnp.dot(h2, wdlwt_ref[...], preferred_element_type=jnp.float32) + wdlb_ref[...]
    pc = pc_ref[...]
    m = jnp.maximum((pc - 2) * _NB, 0)
    bucket = jnp.minimum((m * 1093) >> 15, _NB - 1)
    cols = lax.broadcasted_iota(jnp.int32, (blk, _NB), 1)
    bmask = (cols == bucket).astype(jnp.float32)
    cp_ref[...] = jnp.sum(cp_all * bmask, axis=1, keepdims=True)
    w_ = jnp.sum(wdl_all[:, 0:_NB] * bmask, axis=1, keepdims=True)
    d_ = jnp.sum(wdl_all[:, _NB:2 * _NB] * bmask, axis=1, keepdims=True)
    l_ = jnp.sum(wdl_all[:, 2 * _NB:3 * _NB] * bmask, axis=1, keepdims=True)
    wdl_ref[...] = jnp.concatenate([w_, d_, l_], axis=1)


def _dense_tail(h_pre, piece_count, bias1, fc2_w, fc2_b, cp_w, cp_b, wdl_w, wdl_b,
                interpret=False):
    n = piece_count.shape[0]
    blk = 512
    grid = n // blk
    h = h_pre.shape[1]
    w1t = fc2_w[:, :h].T
    w2t = fc2_w[:, h:].T
    cpwt = cp_w.T
    wdlw_r = wdl_w.reshape(_NB, 3, wdl_w.shape[1]).transpose(1, 0, 2).reshape(3 * _NB, -1)
    wdlwt = wdlw_r.T
    wdlb_r = wdl_b.reshape(_NB, 3).T.reshape(1, 3 * _NB)
    pc2 = piece_count.astype(jnp.int32).reshape(n, 1)

    full = lambda a: pl.BlockSpec(a.shape, lambda i: (0,) * a.ndim)
    cp, wdl = pl.pallas_call(
        _dense_body,
        grid=(grid,),
        in_specs=[
            pl.BlockSpec((blk, h), lambda i: (i, 0)),
            pl.BlockSpec((blk, h), lambda i: (i + n // 512, 0)),
            pl.BlockSpec((blk, 1), lambda i: (i, 0)),
            full(bias1.reshape(1, h)),
            full(w1t),
            full(w2t),
            full(fc2_b.reshape(1, -1)),
            full(cpwt),
            full(cp_b.reshape(1, -1)),
            full(wdlwt),
            full(wdlb_r),
        ],
        out_specs=[
            pl.BlockSpec((blk, 1), lambda i: (i, 0)),
            pl.BlockSpec((blk, 3), lambda i: (i, 0)),
        ],
        out_shape=[
            jax.ShapeDtypeStruct((n, 1), jnp.float32),
            jax.ShapeDtypeStruct((n, 3), jnp.float32),
        ],
        interpret=interpret,
    )(h_pre, h_pre, pc2, bias1.reshape(1, h), w1t, w2t, fc2_b.reshape(1, -1),
      cpwt, cp_b.reshape(1, -1), wdlwt, wdlb_r)
    return cp, wdl


def kernel(x_white, x_black, piece_count, emb, bias1, fc2_w, fc2_b, cp_w, cp_b,
           wdl_w, wdl_b):
    x_all = jnp.concatenate([x_white, x_black], axis=0).astype(jnp.int32)
    sc = _make_sc_bag_sum(x_all.shape[0], x_all.shape[1], emb.shape[1])
    h_pre = sc(x_all.reshape(-1), emb)
    return _dense_tail(h_pre, piece_count, bias1, fc2_w, fc2_b, cp_w, cp_b,
                       wdl_w, wdl_b)
